# baseline (speedup 1.0000x reference)
import functools

import jax
import jax.numpy as jnp
from jax import lax
from jax.experimental import pallas as pl
from jax.experimental.pallas import tpu as pltpu
from jax.experimental.pallas import tpu_sc as plsc

B = 16384
E = 64
NC = 2
NS = 16
NW = NC * NS
BPW = B // NW
CH = 128
NCHUNK = BPW // CH

_mesh = plsc.VectorSubcoreMesh(
    core_axis_name="c", subcore_axis_name="s", num_cores=NC, num_subcores=NS
)


@functools.partial(
    pl.kernel,
    out_type=[
        jax.ShapeDtypeStruct((NW, 16), jnp.float32),
        jax.ShapeDtypeStruct((B // 128, 128), jnp.float32),
    ],
    mesh=_mesh,
    compiler_params=pltpu.CompilerParams(needs_layout_passes=False),
    scratch_types=[
        pltpu.VMEM((NCHUNK, 128), jnp.int32),
        pltpu.VMEM((NCHUNK, 128), jnp.int32),
        pltpu.VMEM((2 * CH * E,), jnp.float32),
        pltpu.VMEM((2 * CH * E,), jnp.float32),
        pltpu.VMEM((BPW * 8,), jnp.float32),
        pltpu.VMEM((BPW * 8,), jnp.float32),
        pltpu.VMEM((NCHUNK, 128), jnp.float32),
        pltpu.VMEM((16,), jnp.float32),
        pltpu.SemaphoreType.DMA,
        pltpu.SemaphoreType.DMA,
        pltpu.SemaphoreType.DMA,
    ],
)
def _sc_gather_dot(
    uidx_hbm, midx_hbm, uemb_hbm, memb_hbm, ubias_hbm, mbias_hbm,
    part_hbm, bsum_hbm,
    uidx_v, midx_v, urows_v, mrows_v, ub_v, mb_v, bsum_v, part_v,
    sem_a, sem_b, sem_bias,
):
    wid = lax.axis_index("s") * NC + lax.axis_index("c")
    pltpu.sync_copy(uidx_hbm.at[pl.ds(wid * NCHUNK, NCHUNK)], uidx_v)
    pltpu.sync_copy(midx_hbm.at[pl.ds(wid * NCHUNK, NCHUNK)], midx_v)

    sems = [sem_a, sem_b]

    def fire_chunk(k):
        boff = (k % 2) * CH
        sem = sems[k % 2]

        def f(g, _):
            ju = uidx_v[k, pl.ds(g * 16, 16)]
            jm = midx_v[k, pl.ds(g * 16, 16)]
            for l in range(16):
                iu = ju[l]
                im = jm[l]
                d = boff + g * 16 + l
                r = k * CH + g * 16 + l
                pltpu.async_copy(
                    uemb_hbm.at[pl.ds(iu * E, E)],
                    urows_v.at[pl.ds(d * E, E)], sem)
                pltpu.async_copy(
                    memb_hbm.at[pl.ds(im * E, E)],
                    mrows_v.at[pl.ds(d * E, E)], sem)
                pltpu.async_copy(
                    ubias_hbm.at[pl.ds((iu // 8) * 8, 8)],
                    ub_v.at[pl.ds(r * 8, 8)], sem_bias)
                pltpu.async_copy(
                    mbias_hbm.at[pl.ds((im // 8) * 8, 8)],
                    mb_v.at[pl.ds(r * 8, 8)], sem_bias)
            return 0

        lax.fori_loop(0, CH // 16, f, 0)

    def drain_rows(k):
        sem = sems[k % 2]

        def d(_, carry):
            pltpu.make_async_copy(
                uemb_hbm.at[pl.ds(0, E)], urows_v.at[pl.ds(0, E)], sem).wait()
            pltpu.make_async_copy(
                uemb_hbm.at[pl.ds(0, E)], urows_v.at[pl.ds(0, E)], sem).wait()
            return carry

        lax.fori_loop(0, CH, d, 0)

    def compute_chunk(k, acc):
        boff = (k % 2) * CH

        def body(rr, a):
            for c in range(E // 16):
                sl = pl.ds((boff + rr) * E + c * 16, 16)
                a = a + urows_v[sl] * mrows_v[sl]
            return a

        return lax.fori_loop(0, CH, body, acc)

    fire_chunk(0)
    acc = jnp.zeros((16,), jnp.float32)
    for k in range(NCHUNK):
        if k + 1 < NCHUNK:
            fire_chunk(k + 1)
        drain_rows(k)
        acc = compute_chunk(k, acc)

    part_v[...] = acc
    pltpu.sync_copy(part_v, part_hbm.at[wid])

    def drain_bias(_, carry):
        pltpu.make_async_copy(
            ubias_hbm.at[pl.ds(0, 8)], ub_v.at[pl.ds(0, 8)], sem_bias).wait()
        pltpu.make_async_copy(
            ubias_hbm.at[pl.ds(0, 8)], ub_v.at[pl.ds(0, 8)], sem_bias).wait()
        return carry

    lax.fori_loop(0, BPW, drain_bias, 0)

    lanes8 = lax.iota(jnp.int32, 16) * 8
    for j in range(NCHUNK):
        for c in range(8):
            sl = pl.ds(c * 16, 16)
            ju = uidx_v[j, sl]
            jm = midx_v[j, sl]
            base = (j * 128 + c * 16) * 8 + lanes8
            uvals = plsc.load_gather(ub_v, [base + (ju % 8)])
            mvals = plsc.load_gather(mb_v, [base + (jm % 8)])
            bsum_v[j, sl] = uvals + mvals
    pltpu.sync_copy(bsum_v, bsum_hbm.at[pl.ds(wid * NCHUNK, NCHUNK)])


def _tc_finish(part_ref, x_ref, o_ref):
    s = jnp.sum(part_ref[...])
    v = x_ref[...] + s
    o_ref[...] = 1.0 / (1.0 + jnp.exp(-v))


def kernel(inputs, user_embedding, user_bias, movie_embedding, movie_bias):
    uidx = inputs[:, 0].reshape(B // 128, 128)
    midx = inputs[:, 1].reshape(B // 128, 128)
    partials, bsum = _sc_gather_dot(
        uidx, midx, user_embedding.reshape(-1), movie_embedding.reshape(-1),
        user_bias.reshape(-1), movie_bias.reshape(-1))
    out = pl.pallas_call(
        _tc_finish,
        out_shape=jax.ShapeDtypeStruct((B // 128, 128), jnp.float32),
    )(partials, bsum)
    return out.reshape(B, 1)


# speedup vs baseline: 1.5249x; 1.5249x over previous
import functools

import jax
import jax.numpy as jnp
from jax import lax
from jax.experimental import pallas as pl
from jax.experimental.pallas import tpu as pltpu
from jax.experimental.pallas import tpu_sc as plsc

B = 16384
E = 64
V = 100000
NC = 2
NS = 16
NW = NC * NS
BPW = B // NW
EPW = E // NW
QR = 32
NQ = (B // 128) // QR

_mesh = plsc.VectorSubcoreMesh(
    core_axis_name="c", subcore_axis_name="s", num_cores=NC, num_subcores=NS
)


@functools.partial(
    pl.kernel,
    out_type=[
        jax.ShapeDtypeStruct((NW, 16), jnp.float32),
        jax.ShapeDtypeStruct((B // 128, 128), jnp.float32),
    ],
    mesh=_mesh,
    compiler_params=pltpu.CompilerParams(needs_layout_passes=False),
    scratch_types=[
        pltpu.VMEM((1, V), jnp.float32),
        pltpu.VMEM((B,), jnp.float32),
        pltpu.VMEM((QR, 128), jnp.int32),
        pltpu.VMEM((NW // 8, 128), jnp.int32),
        pltpu.VMEM((NW // 8, 128), jnp.int32),
        pltpu.VMEM((BPW * 8,), jnp.float32),
        pltpu.VMEM((BPW * 8,), jnp.float32),
        pltpu.VMEM((NW // 8, 128), jnp.float32),
        pltpu.VMEM((16,), jnp.float32),
        pltpu.SemaphoreType.DMA,
        pltpu.SemaphoreType.DMA,
    ],
)
def _sc_coldot(
    uidx_hbm, midx_hbm, uembt_hbm, membt_hbm, ubias_hbm, mbias_hbm,
    part_hbm, bsum_hbm,
    row_v, uv_v, idx_v, biu_v, bim_v, ub_v, mb_v, bsum_v, part_v,
    sem_row, sem_bias,
):
    wid = lax.axis_index("s") * NC + lax.axis_index("c")
    nch = NW // 8

    pltpu.sync_copy(uidx_hbm.at[pl.ds(wid * nch, nch)], biu_v)
    pltpu.sync_copy(midx_hbm.at[pl.ds(wid * nch, nch)], bim_v)

    def bias_fire(g, _):
        ju = biu_v[g // 8, pl.ds((g % 8) * 16, 16)]
        jm = bim_v[g // 8, pl.ds((g % 8) * 16, 16)]
        for l in range(16):
            iu = ju[l]
            im = jm[l]
            r = g * 16 + l
            pltpu.async_copy(
                ubias_hbm.at[pl.ds((iu // 8) * 8, 8)],
                ub_v.at[pl.ds(r * 8, 8)], sem_bias)
            pltpu.async_copy(
                mbias_hbm.at[pl.ds((im // 8) * 8, 8)],
                mb_v.at[pl.ds(r * 8, 8)], sem_bias)
        return 0

    lax.fori_loop(0, BPW // 16, bias_fire, 0)

    z16 = jnp.zeros((16,), jnp.int32)
    acc = jnp.zeros((16,), jnp.float32)
    for cc in range(EPW):
        e = wid * EPW + cc
        pltpu.sync_copy(uembt_hbm.at[pl.ds(e, 1), pl.ds(0, V)], row_v)
        for q in range(NQ):
            pltpu.sync_copy(uidx_hbm.at[pl.ds(q * QR, QR)], idx_v)

            def gu(i, _, q=q):
                iv = idx_v[i // 8, pl.ds((i % 8) * 16, 16)]
                vals = plsc.load_gather(row_v, [z16, iv])
                uv_v[pl.ds(q * QR * 128 + i * 16, 16)] = vals
                return 0

            lax.fori_loop(0, QR * 8, gu, 0)
        pltpu.sync_copy(membt_hbm.at[pl.ds(e, 1), pl.ds(0, V)], row_v)
        for q in range(NQ):
            pltpu.sync_copy(midx_hbm.at[pl.ds(q * QR, QR)], idx_v)

            def gm(i, a, q=q):
                jv = idx_v[i // 8, pl.ds((i % 8) * 16, 16)]
                vals = plsc.load_gather(row_v, [z16, jv])
                return a + uv_v[pl.ds(q * QR * 128 + i * 16, 16)] * vals

            acc = lax.fori_loop(0, QR * 8, gm, acc)

    part_v[...] = acc
    pltpu.sync_copy(part_v, part_hbm.at[wid])

    def drain_bias(_, carry):
        pltpu.make_async_copy(
            ubias_hbm.at[pl.ds(0, 8)], ub_v.at[pl.ds(0, 8)], sem_bias).wait()
        pltpu.make_async_copy(
            ubias_hbm.at[pl.ds(0, 8)], ub_v.at[pl.ds(0, 8)], sem_bias).wait()
        return carry

    lax.fori_loop(0, BPW, drain_bias, 0)

    lanes8 = lax.iota(jnp.int32, 16) * 8
    for j in range(nch):
        for c in range(8):
            sl = pl.ds(c * 16, 16)
            ju = biu_v[j, sl]
            jm = bim_v[j, sl]
            base = (j * 128 + c * 16) * 8 + lanes8
            uvals = plsc.load_gather(ub_v, [base + (ju % 8)])
            mvals = plsc.load_gather(mb_v, [base + (jm % 8)])
            bsum_v[j, sl] = uvals + mvals
    pltpu.sync_copy(bsum_v, bsum_hbm.at[pl.ds(wid * nch, nch)])


def _tc_finish(part_ref, x_ref, o_ref):
    s = jnp.sum(part_ref[...])
    v = x_ref[...] + s
    o_ref[...] = 1.0 / (1.0 + jnp.exp(-v))


def kernel(inputs, user_embedding, user_bias, movie_embedding, movie_bias):
    uidx = inputs[:, 0].reshape(B // 128, 128)
    midx = inputs[:, 1].reshape(B // 128, 128)
    partials, bsum = _sc_coldot(
        uidx, midx, user_embedding.T, movie_embedding.T,
        user_bias.reshape(-1), movie_bias.reshape(-1))
    out = pl.pallas_call(
        _tc_finish,
        out_shape=jax.ShapeDtypeStruct((B // 128, 128), jnp.float32),
    )(partials, bsum)
    return out.reshape(B, 1)


# speedup vs baseline: 1.8418x; 1.2078x over previous
import functools

import jax
import jax.numpy as jnp
from jax import lax
from jax.experimental import pallas as pl
from jax.experimental.pallas import tpu as pltpu
from jax.experimental.pallas import tpu_sc as plsc

B = 16384
E = 64
V = 100000
NC = 2
NS = 16
NW = NC * NS
BPW = B // NW
EPW = E // NW
QR = 16
NQ = (B // 128) // QR

_mesh = plsc.VectorSubcoreMesh(
    core_axis_name="c", subcore_axis_name="s", num_cores=NC, num_subcores=NS
)


@functools.partial(
    pl.kernel,
    out_type=[
        jax.ShapeDtypeStruct((NW, 16), jnp.float32),
        jax.ShapeDtypeStruct((B // 128, 128), jnp.float32),
    ],
    mesh=_mesh,
    compiler_params=pltpu.CompilerParams(needs_layout_passes=False),
    scratch_types=[
        pltpu.VMEM((1, V), jnp.float32),
        pltpu.VMEM((B,), jnp.float32),
        pltpu.VMEM((2, QR, 128), jnp.int32),
        pltpu.VMEM((NW // 8, 128), jnp.int32),
        pltpu.VMEM((NW // 8, 128), jnp.int32),
        pltpu.VMEM((BPW,), jnp.float32),
        pltpu.VMEM((BPW,), jnp.float32),
        pltpu.VMEM((NW // 8, 128), jnp.float32),
        pltpu.VMEM((16,), jnp.float32),
        pltpu.VMEM_SHARED((V,), jnp.float32),
        pltpu.SemaphoreType.DMA,
        pltpu.SemaphoreType.DMA,
    ],
)
def _sc_coldot(
    uidx_hbm, midx_hbm, uembt_hbm, membt_hbm, ubias_hbm, mbias_hbm,
    part_hbm, bsum_hbm,
    row_v, uv_v, idx_v, biu_v, bim_v, ub_v, mb_v, bsum_v, part_v,
    sb_s,
    sem_idx, sem_bias,
):
    sid = lax.axis_index("s")
    wid = sid * NC + lax.axis_index("c")
    nch = NW // 8

    @pl.when(sid == 0)
    def _():
        pltpu.sync_copy(ubias_hbm, sb_s)

    pltpu.sync_copy(uidx_hbm.at[pl.ds(wid * nch, nch)], biu_v)
    pltpu.sync_copy(midx_hbm.at[pl.ds(wid * nch, nch)], bim_v)
    plsc.subcore_barrier()

    def fire_bias(idx_ref, dst_ref):
        return [pltpu.async_copy(
            sb_s.at[idx_ref.at[j]], dst_ref.at[pl.ds(j * 128, 128)], sem_bias)
            for j in range(nch)]

    bias_copies = fire_bias(biu_v, ub_v)

    z16 = jnp.zeros((16,), jnp.int32)
    acc = jnp.zeros((16,), jnp.float32)

    def stage_idx(table_hbm, q, buf):
        return pltpu.async_copy(
            table_hbm.at[pl.ds(q * QR, QR)], idx_v.at[buf], sem_idx)

    for cc in range(EPW):
        if cc == 1:
            for c in bias_copies:
                c.wait()
            plsc.subcore_barrier()

            @pl.when(sid == 0)
            def _():
                pltpu.sync_copy(mbias_hbm, sb_s)

            plsc.subcore_barrier()
            bias_copies = fire_bias(bim_v, mb_v)
        e = wid * EPW + cc
        for half, (tab, idxtab) in enumerate(
                ((uembt_hbm, uidx_hbm), (membt_hbm, midx_hbm))):
            cp = stage_idx(idxtab, 0, 0)
            pltpu.sync_copy(tab.at[pl.ds(e, 1), pl.ds(0, V)], row_v)
            for q in range(NQ):
                if q + 1 < NQ:
                    nxt = stage_idx(idxtab, q + 1, (q + 1) % 2)
                cp.wait()

                if half == 0:
                    def gu(i, _, q=q, buf=q % 2):
                        for u in range(4):
                            iv = idx_v[buf, (i * 4 + u) // 8,
                                       pl.ds(((i * 4 + u) % 8) * 16, 16)]
                            vals = plsc.load_gather(row_v, [z16, iv])
                            uv_v[pl.ds(q * QR * 128 + (i * 4 + u) * 16, 16)] \
                                = vals
                        return 0

                    lax.fori_loop(0, QR * 2, gu, 0)
                else:
                    def gm(i, a, q=q, buf=q % 2):
                        for u in range(4):
                            jv = idx_v[buf, (i * 4 + u) // 8,
                                       pl.ds(((i * 4 + u) % 8) * 16, 16)]
                            vals = plsc.load_gather(row_v, [z16, jv])
                            a = a + uv_v[pl.ds(
                                q * QR * 128 + (i * 4 + u) * 16, 16)] * vals
                        return a

                    acc = lax.fori_loop(0, QR * 2, gm, acc)
                if q + 1 < NQ:
                    cp = nxt

    part_v[...] = acc
    pltpu.sync_copy(part_v, part_hbm.at[wid])

    for c in bias_copies:
        c.wait()
    for j in range(nch):
        for c in range(8):
            sl = pl.ds(j * 128 + c * 16, 16)
            bsum_v[j, pl.ds(c * 16, 16)] = ub_v[sl] + mb_v[sl]
    pltpu.sync_copy(bsum_v, bsum_hbm.at[pl.ds(wid * nch, nch)])


def _tc_finish(part_ref, x_ref, o_ref):
    s = jnp.sum(part_ref[...])
    v = x_ref[...] + s
    o_ref[...] = 1.0 / (1.0 + jnp.exp(-v))


def kernel(inputs, user_embedding, user_bias, movie_embedding, movie_bias):
    uidx = inputs[:, 0].reshape(B // 128, 128)
    midx = inputs[:, 1].reshape(B // 128, 128)
    partials, bsum = _sc_coldot(
        uidx, midx, user_embedding.T, movie_embedding.T,
        user_bias.reshape(-1), movie_bias.reshape(-1))
    out = pl.pallas_call(
        _tc_finish,
        out_shape=jax.ShapeDtypeStruct((B // 128, 128), jnp.float32),
    )(partials, bsum)
    return out.reshape(B, 1)
